# initial kernel scaffold (unmeasured)
import jax
import jax.numpy as jnp
from jax import lax
from jax.experimental import pallas as pl
from jax.experimental.pallas import tpu as pltpu

N_DEV = 32
M = 2048
N = 2048
CHUNK = M // N_DEV


def kernel(A, B):
    def body(a_ref, b_ref, out_ref, rs_recv,
             rs_send_sems, rs_recv_sems, ag_send_sems, ag_recv_sems):
        p = lax.axis_index("i")
        left = lax.rem(p + N_DEV - 1, N_DEV)
        right = lax.rem(p + 1, N_DEV)

        barrier_sem = pltpu.get_barrier_semaphore()
        for nbr in (left, right):
            pl.semaphore_signal(
                barrier_sem, inc=1,
                device_id=(nbr,), device_id_type=pl.DeviceIdType.MESH,
            )
        pl.semaphore_wait(barrier_sem, 2)

        out_ref[:, :] = jnp.dot(
            a_ref[:, :], b_ref[:, :], preferred_element_type=jnp.float32
        )

        for h in range(N_DEV - 1):
            send_c = lax.rem(p - h + N_DEV, N_DEV)
            recv_c = lax.rem(p - h - 1 + N_DEV, N_DEV)
            rdma = pltpu.make_async_remote_copy(
                src_ref=out_ref.at[pl.ds(send_c * CHUNK, CHUNK), :],
                dst_ref=rs_recv.at[h],
                send_sem=rs_send_sems.at[h],
                recv_sem=rs_recv_sems.at[h],
                device_id=(right,),
                device_id_type=pl.DeviceIdType.MESH,
            )
            rdma.start()
            rdma.wait()
            rows = pl.ds(recv_c * CHUNK, CHUNK)
            out_ref[rows, :] = out_ref[rows, :] + rs_recv[h, :, :]

        cstar = lax.rem(p + 1, N_DEV)
        rows = pl.ds(cstar * CHUNK, CHUNK)
        z = out_ref[rows, :]
        out_ref[rows, :] = z / (1.0 + jnp.exp(-z))

        for h in range(N_DEV - 1):
            send_c = lax.rem(p + 1 - h + N_DEV, N_DEV)
            src = out_ref.at[pl.ds(send_c * CHUNK, CHUNK), :]
            rdma = pltpu.make_async_remote_copy(
                src_ref=src,
                dst_ref=src,
                send_sem=ag_send_sems.at[h],
                recv_sem=ag_recv_sems.at[h],
                device_id=(right,),
                device_id_type=pl.DeviceIdType.MESH,
            )
            rdma.start()
            rdma.wait()

    return pl.pallas_call(
        body,
        out_shape=jax.ShapeDtypeStruct((M, N), jnp.float32),
        in_specs=[
            pl.BlockSpec(memory_space=pltpu.VMEM),
            pl.BlockSpec(memory_space=pltpu.VMEM),
        ],
        out_specs=pl.BlockSpec(memory_space=pltpu.VMEM),
        scratch_shapes=[
            pltpu.VMEM((N_DEV - 1, CHUNK, N), jnp.float32),
            pltpu.SemaphoreType.DMA((N_DEV - 1,)),
            pltpu.SemaphoreType.DMA((N_DEV - 1,)),
            pltpu.SemaphoreType.DMA((N_DEV - 1,)),
            pltpu.SemaphoreType.DMA((N_DEV - 1,)),
        ],
        compiler_params=pltpu.CompilerParams(collective_id=0),
    )(A, B)


# baseline (device time: 503246 ns/iter reference)
import jax
import jax.numpy as jnp
from jax import lax
from jax.experimental import pallas as pl
from jax.experimental.pallas import tpu as pltpu

N_DEV = 32
M = 2048
N = 2048
CHUNK = M // N_DEV


def kernel(A, B):
    def body(a_ref, b_ref, out_ref, rs_recv,
             rs_send_sems, rs_recv_sems, ag_send_sems, ag_recv_sems):
        p = lax.axis_index("i")
        left = lax.rem(p + N_DEV - 1, N_DEV)
        right = lax.rem(p + 1, N_DEV)

        barrier_sem = pltpu.get_barrier_semaphore()
        for nbr in (left, right):
            pl.semaphore_signal(
                barrier_sem, inc=1,
                device_id=(nbr,), device_id_type=pl.DeviceIdType.MESH,
            )
        pl.semaphore_wait(barrier_sem, 2)

        out_ref[:, :] = jnp.dot(
            a_ref[:, :], b_ref[:, :], preferred_element_type=jnp.float32
        )

        for h in range(N_DEV - 1):
            send_c = lax.rem(p - h + N_DEV, N_DEV)
            recv_c = lax.rem(p - h - 1 + N_DEV, N_DEV)
            rdma = pltpu.make_async_remote_copy(
                src_ref=out_ref.at[pl.ds(send_c * CHUNK, CHUNK), :],
                dst_ref=rs_recv.at[h],
                send_sem=rs_send_sems.at[h],
                recv_sem=rs_recv_sems.at[h],
                device_id=(right,),
                device_id_type=pl.DeviceIdType.MESH,
            )
            rdma.start()
            rdma.wait()
            rows = pl.ds(recv_c * CHUNK, CHUNK)
            out_ref[rows, :] = out_ref[rows, :] + rs_recv[h, :, :]

        cstar = lax.rem(p + 1, N_DEV)
        rows = pl.ds(cstar * CHUNK, CHUNK)
        z = out_ref[rows, :]
        out_ref[rows, :] = z / (1.0 + jnp.exp(-z))

        for h in range(N_DEV - 1):
            send_c = lax.rem(p + 1 - h + N_DEV, N_DEV)
            src = out_ref.at[pl.ds(send_c * CHUNK, CHUNK), :]
            rdma = pltpu.make_async_remote_copy(
                src_ref=src,
                dst_ref=src,
                send_sem=ag_send_sems.at[h],
                recv_sem=ag_recv_sems.at[h],
                device_id=(right,),
                device_id_type=pl.DeviceIdType.MESH,
            )
            rdma.start()
            rdma.wait()

    return pl.pallas_call(
        body,
        out_shape=jax.ShapeDtypeStruct((M, N), jnp.float32),
        in_specs=[
            pl.BlockSpec(memory_space=pltpu.VMEM),
            pl.BlockSpec(memory_space=pltpu.VMEM),
        ],
        out_specs=pl.BlockSpec(memory_space=pltpu.VMEM),
        scratch_shapes=[
            pltpu.VMEM((N_DEV - 1, CHUNK, N), jnp.float32),
            pltpu.SemaphoreType.DMA((N_DEV - 1,)),
            pltpu.SemaphoreType.DMA((N_DEV - 1,)),
            pltpu.SemaphoreType.DMA((N_DEV - 1,)),
            pltpu.SemaphoreType.DMA((N_DEV - 1,)),
        ],
        compiler_params=pltpu.CompilerParams(
            collective_id=0, vmem_limit_bytes=100 * 1024 * 1024
        ),
    )(A, B)


# device time: 474222 ns/iter; 1.0612x vs baseline; 1.0612x over previous
import jax
import jax.numpy as jnp
from jax import lax
from jax.experimental import pallas as pl
from jax.experimental.pallas import tpu as pltpu

N_DEV = 32
M = 2048
N = 2048
HALF = M // 2
CHUNK = HALF // N_DEV
LAST = N_DEV - 4


def kernel(A, B):
    def body(a_ref, b_ref, out_ref, buf_cw, buf_ccw,
             rs_snd_cw, rs_rcv_cw, ag_snd_cw, ag_rcv_cw,
             rs_snd_ccw, rs_rcv_ccw, ag_snd_ccw, ag_rcv_ccw,
             rs_cred_cw, rs_cred_ccw, ag_cred_cw, ag_cred_ccw):
        p = lax.axis_index("i")
        left = lax.rem(p + N_DEV - 1, N_DEV)
        right = lax.rem(p + 1, N_DEV)

        barrier_sem = pltpu.get_barrier_semaphore()
        for nbr in (left, right):
            pl.semaphore_signal(
                barrier_sem, inc=1,
                device_id=(nbr,), device_id_type=pl.DeviceIdType.MESH,
            )
        pl.semaphore_wait(barrier_sem, 2)

        out_ref[:, :] = jnp.dot(
            a_ref[:, :], b_ref[:, :], preferred_element_type=jnp.float32
        )

        def top(c):
            return pl.ds(c * CHUNK, CHUNK)

        def bot(c):
            return pl.ds(HALF + c * CHUNK, CHUNK)

        def rs_hop(h, carry):
            send_cw = lax.rem(p - h + N_DEV, N_DEV)
            recv_cw = lax.rem(p - h - 1 + N_DEV, N_DEV)
            send_ccw = lax.rem(p + h, N_DEV)
            recv_ccw = lax.rem(p + h + 1, N_DEV)

            def do(s):
                @pl.when(h >= 2)
                def _():
                    pl.semaphore_wait(rs_cred_cw.at[s], 1)
                    pl.semaphore_wait(rs_cred_ccw.at[s], 1)

                rdma_cw = pltpu.make_async_remote_copy(
                    src_ref=out_ref.at[top(send_cw), :],
                    dst_ref=buf_cw.at[s],
                    send_sem=rs_snd_cw.at[s],
                    recv_sem=rs_rcv_cw.at[s],
                    device_id=(right,),
                    device_id_type=pl.DeviceIdType.MESH,
                )
                rdma_ccw = pltpu.make_async_remote_copy(
                    src_ref=out_ref.at[bot(send_ccw), :],
                    dst_ref=buf_ccw.at[s],
                    send_sem=rs_snd_ccw.at[s],
                    recv_sem=rs_rcv_ccw.at[s],
                    device_id=(left,),
                    device_id_type=pl.DeviceIdType.MESH,
                )
                rdma_cw.start()
                rdma_ccw.start()
                rdma_cw.wait()
                rdma_ccw.wait()
                rows = top(recv_cw)
                out_ref[rows, :] = out_ref[rows, :] + buf_cw[s, :, :]
                rows = bot(recv_ccw)
                out_ref[rows, :] = out_ref[rows, :] + buf_ccw[s, :, :]

                @pl.when(h <= LAST)
                def _():
                    pl.semaphore_signal(
                        rs_cred_cw.at[s], inc=1,
                        device_id=(left,), device_id_type=pl.DeviceIdType.MESH,
                    )
                    pl.semaphore_signal(
                        rs_cred_ccw.at[s], inc=1,
                        device_id=(right,), device_id_type=pl.DeviceIdType.MESH,
                    )

            @pl.when(lax.rem(h, 2) == 0)
            def _():
                do(0)

            @pl.when(lax.rem(h, 2) == 1)
            def _():
                do(1)

            return carry

        lax.fori_loop(0, N_DEV - 1, rs_hop, 0)

        for rows in (top(lax.rem(p + 1, N_DEV)),
                     bot(lax.rem(p + N_DEV - 1, N_DEV))):
            z = out_ref[rows, :]
            out_ref[rows, :] = z / (1.0 + jnp.exp(-z))

        def ag_hop(h, carry):
            send_cw = lax.rem(p + 1 - h + N_DEV, N_DEV)
            send_ccw = lax.rem(p - 1 + h + N_DEV, N_DEV)

            def do(s):
                @pl.when(h >= 2)
                def _():
                    pl.semaphore_wait(ag_cred_cw.at[s], 1)
                    pl.semaphore_wait(ag_cred_ccw.at[s], 1)

                src_cw = out_ref.at[top(send_cw), :]
                src_ccw = out_ref.at[bot(send_ccw), :]
                rdma_cw = pltpu.make_async_remote_copy(
                    src_ref=src_cw, dst_ref=src_cw,
                    send_sem=ag_snd_cw.at[s],
                    recv_sem=ag_rcv_cw.at[s],
                    device_id=(right,),
                    device_id_type=pl.DeviceIdType.MESH,
                )
                rdma_ccw = pltpu.make_async_remote_copy(
                    src_ref=src_ccw, dst_ref=src_ccw,
                    send_sem=ag_snd_ccw.at[s],
                    recv_sem=ag_rcv_ccw.at[s],
                    device_id=(left,),
                    device_id_type=pl.DeviceIdType.MESH,
                )
                rdma_cw.start()
                rdma_ccw.start()
                rdma_cw.wait()
                rdma_ccw.wait()

                @pl.when(h <= LAST)
                def _():
                    pl.semaphore_signal(
                        ag_cred_cw.at[s], inc=1,
                        device_id=(left,), device_id_type=pl.DeviceIdType.MESH,
                    )
                    pl.semaphore_signal(
                        ag_cred_ccw.at[s], inc=1,
                        device_id=(right,), device_id_type=pl.DeviceIdType.MESH,
                    )

            @pl.when(lax.rem(h, 2) == 0)
            def _():
                do(0)

            @pl.when(lax.rem(h, 2) == 1)
            def _():
                do(1)

            return carry

        lax.fori_loop(0, N_DEV - 1, ag_hop, 0)

    return pl.pallas_call(
        body,
        out_shape=jax.ShapeDtypeStruct((M, N), jnp.float32),
        in_specs=[
            pl.BlockSpec(memory_space=pltpu.VMEM),
            pl.BlockSpec(memory_space=pltpu.VMEM),
        ],
        out_specs=pl.BlockSpec(memory_space=pltpu.VMEM),
        scratch_shapes=[
            pltpu.VMEM((2, CHUNK, N), jnp.float32),
            pltpu.VMEM((2, CHUNK, N), jnp.float32),
            pltpu.SemaphoreType.DMA((2,)),
            pltpu.SemaphoreType.DMA((2,)),
            pltpu.SemaphoreType.DMA((2,)),
            pltpu.SemaphoreType.DMA((2,)),
            pltpu.SemaphoreType.DMA((2,)),
            pltpu.SemaphoreType.DMA((2,)),
            pltpu.SemaphoreType.DMA((2,)),
            pltpu.SemaphoreType.DMA((2,)),
            pltpu.SemaphoreType.REGULAR((2,)),
            pltpu.SemaphoreType.REGULAR((2,)),
            pltpu.SemaphoreType.REGULAR((2,)),
            pltpu.SemaphoreType.REGULAR((2,)),
        ],
        compiler_params=pltpu.CompilerParams(
            collective_id=0, vmem_limit_bytes=100 * 1024 * 1024
        ),
    )(A, B)


# device time: 325552 ns/iter; 1.5458x vs baseline; 1.4567x over previous
import jax
import jax.numpy as jnp
from jax import lax
from jax.experimental import pallas as pl
from jax.experimental.pallas import tpu as pltpu

N_DEV = 32
M = 2048
N = 2048
HALF = M // 2
CHUNK = HALF // N_DEV
LAST = N_DEV - 4

RING = [0, 1, 2, 3, 4, 5, 6, 7, 15, 14, 13, 12, 11, 10, 18, 19,
        20, 21, 22, 23, 31, 30, 29, 28, 27, 26, 25, 24, 16, 17, 9, 8]
INV = [0] * N_DEV
for _k, _l in enumerate(RING):
    INV[_l] = _k


def kernel(A, B):
    p = lax.axis_index("i")
    ring = jnp.asarray(RING, jnp.int32)
    inv = jnp.asarray(INV, jnp.int32)
    q = inv[p]
    right = ring[lax.rem(q + 1, N_DEV)]
    left = ring[lax.rem(q + N_DEV - 1, N_DEV)]
    qlr = jnp.stack([q, left, right]).astype(jnp.int32)

    def body(a_ref, b_ref, qlr_ref, out_ref, buf_cw, buf_ccw,
             rs_snd_cw, rs_rcv_cw, ag_snd_cw, ag_rcv_cw,
             rs_snd_ccw, rs_rcv_ccw, ag_snd_ccw, ag_rcv_ccw,
             rs_cred_cw, rs_cred_ccw, ag_cred_cw, ag_cred_ccw):
        q = qlr_ref[0]
        left = qlr_ref[1]
        right = qlr_ref[2]

        barrier_sem = pltpu.get_barrier_semaphore()
        for nbr in (left, right):
            pl.semaphore_signal(
                barrier_sem, inc=1,
                device_id=(nbr,), device_id_type=pl.DeviceIdType.MESH,
            )
        pl.semaphore_wait(barrier_sem, 2)

        out_ref[:, :] = jnp.dot(
            a_ref[:, :], b_ref[:, :], preferred_element_type=jnp.float32
        )

        def top(c):
            return pl.ds(c * CHUNK, CHUNK)

        def bot(c):
            return pl.ds(HALF + c * CHUNK, CHUNK)

        def rs_hop(h, carry):
            send_cw = lax.rem(q - h + N_DEV, N_DEV)
            recv_cw = lax.rem(q - h - 1 + N_DEV, N_DEV)
            send_ccw = lax.rem(q + h, N_DEV)
            recv_ccw = lax.rem(q + h + 1, N_DEV)

            def do(s):
                @pl.when(h >= 2)
                def _():
                    pl.semaphore_wait(rs_cred_cw.at[s], 1)
                    pl.semaphore_wait(rs_cred_ccw.at[s], 1)

                rdma_cw = pltpu.make_async_remote_copy(
                    src_ref=out_ref.at[top(send_cw), :],
                    dst_ref=buf_cw.at[s],
                    send_sem=rs_snd_cw.at[s],
                    recv_sem=rs_rcv_cw.at[s],
                    device_id=(right,),
                    device_id_type=pl.DeviceIdType.MESH,
                )
                rdma_ccw = pltpu.make_async_remote_copy(
                    src_ref=out_ref.at[bot(send_ccw), :],
                    dst_ref=buf_ccw.at[s],
                    send_sem=rs_snd_ccw.at[s],
                    recv_sem=rs_rcv_ccw.at[s],
                    device_id=(left,),
                    device_id_type=pl.DeviceIdType.MESH,
                )
                rdma_cw.start()
                rdma_ccw.start()
                rdma_cw.wait_recv()
                rows = top(recv_cw)
                out_ref[rows, :] = out_ref[rows, :] + buf_cw[s, :, :]

                @pl.when(h <= LAST)
                def _():
                    pl.semaphore_signal(
                        rs_cred_cw.at[s], inc=1,
                        device_id=(left,), device_id_type=pl.DeviceIdType.MESH,
                    )

                rdma_ccw.wait_recv()
                rows = bot(recv_ccw)
                out_ref[rows, :] = out_ref[rows, :] + buf_ccw[s, :, :]

                @pl.when(h <= LAST)
                def _():
                    pl.semaphore_signal(
                        rs_cred_ccw.at[s], inc=1,
                        device_id=(right,), device_id_type=pl.DeviceIdType.MESH,
                    )

                rdma_cw.wait_send()
                rdma_ccw.wait_send()

            @pl.when(lax.rem(h, 2) == 0)
            def _():
                do(0)

            @pl.when(lax.rem(h, 2) == 1)
            def _():
                do(1)

            return carry

        lax.fori_loop(0, N_DEV - 1, rs_hop, 0)

        for rows in (top(lax.rem(q + 1, N_DEV)),
                     bot(lax.rem(q + N_DEV - 1, N_DEV))):
            z = out_ref[rows, :]
            out_ref[rows, :] = z / (1.0 + jnp.exp(-z))

        def ag_hop(h, carry):
            send_cw = lax.rem(q + 1 - h + N_DEV, N_DEV)
            send_ccw = lax.rem(q - 1 + h + N_DEV, N_DEV)

            def do(s):
                @pl.when(h >= 2)
                def _():
                    pl.semaphore_wait(ag_cred_cw.at[s], 1)
                    pl.semaphore_wait(ag_cred_ccw.at[s], 1)

                src_cw = out_ref.at[top(send_cw), :]
                src_ccw = out_ref.at[bot(send_ccw), :]
                rdma_cw = pltpu.make_async_remote_copy(
                    src_ref=src_cw, dst_ref=src_cw,
                    send_sem=ag_snd_cw.at[s],
                    recv_sem=ag_rcv_cw.at[s],
                    device_id=(right,),
                    device_id_type=pl.DeviceIdType.MESH,
                )
                rdma_ccw = pltpu.make_async_remote_copy(
                    src_ref=src_ccw, dst_ref=src_ccw,
                    send_sem=ag_snd_ccw.at[s],
                    recv_sem=ag_rcv_ccw.at[s],
                    device_id=(left,),
                    device_id_type=pl.DeviceIdType.MESH,
                )
                rdma_cw.start()
                rdma_ccw.start()
                rdma_cw.wait_recv()

                @pl.when(h <= LAST)
                def _():
                    pl.semaphore_signal(
                        ag_cred_cw.at[s], inc=1,
                        device_id=(left,), device_id_type=pl.DeviceIdType.MESH,
                    )

                rdma_ccw.wait_recv()

                @pl.when(h <= LAST)
                def _():
                    pl.semaphore_signal(
                        ag_cred_ccw.at[s], inc=1,
                        device_id=(right,), device_id_type=pl.DeviceIdType.MESH,
                    )

                rdma_cw.wait_send()
                rdma_ccw.wait_send()

            @pl.when(lax.rem(h, 2) == 0)
            def _():
                do(0)

            @pl.when(lax.rem(h, 2) == 1)
            def _():
                do(1)

            return carry

        lax.fori_loop(0, N_DEV - 1, ag_hop, 0)

    return pl.pallas_call(
        body,
        out_shape=jax.ShapeDtypeStruct((M, N), jnp.float32),
        in_specs=[
            pl.BlockSpec(memory_space=pltpu.VMEM),
            pl.BlockSpec(memory_space=pltpu.VMEM),
            pl.BlockSpec(memory_space=pltpu.SMEM),
        ],
        out_specs=pl.BlockSpec(memory_space=pltpu.VMEM),
        scratch_shapes=[
            pltpu.VMEM((2, CHUNK, N), jnp.float32),
            pltpu.VMEM((2, CHUNK, N), jnp.float32),
            pltpu.SemaphoreType.DMA((2,)),
            pltpu.SemaphoreType.DMA((2,)),
            pltpu.SemaphoreType.DMA((2,)),
            pltpu.SemaphoreType.DMA((2,)),
            pltpu.SemaphoreType.DMA((2,)),
            pltpu.SemaphoreType.DMA((2,)),
            pltpu.SemaphoreType.DMA((2,)),
            pltpu.SemaphoreType.DMA((2,)),
            pltpu.SemaphoreType.REGULAR((2,)),
            pltpu.SemaphoreType.REGULAR((2,)),
            pltpu.SemaphoreType.REGULAR((2,)),
            pltpu.SemaphoreType.REGULAR((2,)),
        ],
        compiler_params=pltpu.CompilerParams(
            collective_id=0, vmem_limit_bytes=100 * 1024 * 1024
        ),
    )(A, B, qlr)
